# baseline (speedup 1.0000x reference)
import functools

import jax
import jax.numpy as jnp
from jax import lax
from jax.experimental import pallas as pl
from jax.experimental.pallas import tpu as pltpu
from jax.experimental.pallas import tpu_sc as plsc

TAU = 0.8
ALPHA = 0.3
NC, NS, L = 2, 16, 16
NW = NC * NS
GP = 4
UNROLL = 8


@functools.lru_cache(maxsize=None)
def _build_sc_loss(bt: int, v: int):
    tpw = bt // NW
    ngp = tpw // GP
    pairs = ngp // 2
    mesh = plsc.VectorSubcoreMesh(
        core_axis_name="c", subcore_axis_name="s",
        num_cores=NC, num_subcores=NS)

    @functools.partial(
        pl.kernel,
        out_type=(
            jax.ShapeDtypeStruct((bt, L), jnp.float32),
            jax.ShapeDtypeStruct((bt, L), jnp.float32),
        ),
        mesh=mesh,
        scratch_types=[
            pltpu.VMEM((ngp, GP), jnp.int32),
            pltpu.VMEM((GP, v), jnp.float32),
            pltpu.VMEM((GP, v), jnp.float32),
            pltpu.VMEM((GP * v,), jnp.float32),
            pltpu.VMEM((GP * v,), jnp.float32),
            pltpu.VMEM((tpw, L), jnp.float32),
            pltpu.VMEM((tpw, L), jnp.float32),
            pltpu.SemaphoreType.DMA,
            pltpu.SemaphoreType.DMA,
            pltpu.SemaphoreType.DMA,
            pltpu.SemaphoreType.DMA,
        ],
    )
    def sc_loss(x_hbm, tgt2_hbm, dist_hbm,
                esum_hbm, dot_hbm,
                tgt2_v, e0, e1, x0, x1, es_all, dt_all,
                sem_e0, sem_e1, sem_x0, sem_x1):
        wid = lax.axis_index("c") * NS + lax.axis_index("s")
        base = wid * tpw
        base_g = pl.multiple_of(wid * ngp, 8)
        inv_tau = 1.0 / TAU
        pltpu.sync_copy(tgt2_hbm.at[pl.ds(base_g, ngp)], tgt2_v)
        zero = jnp.zeros((L,), jnp.float32)

        def issue(g, ebuf, xbuf, sem_e, sem_x):
            pltpu.async_copy(dist_hbm.at[tgt2_v.at[g]], ebuf, sem_e)
            for r in range(GP):
                pltpu.async_copy(x_hbm.at[base + g * GP + r],
                                 xbuf.at[pl.ds(r * v, v)], sem_x)

        def drain(ebuf, xbuf, sem_e, sem_x):
            pltpu.make_async_copy(dist_hbm.at[tgt2_v.at[0]],
                                  ebuf, sem_e).wait()
            for r in range(GP):
                pltpu.make_async_copy(x_hbm.at[0],
                                      xbuf.at[pl.ds(r * v, v)], sem_x).wait()

        def compute(g, ebuf, xbuf):
            for r in range(GP):
                tok = g * GP + r

                def col_body(i, c3):
                    v_e, v_p = c3
                    for u in range(UNROLL):
                        d = ebuf[r, pl.ds((i * UNROLL + u) * L, L)]
                        x = xbuf[pl.ds(r * v + (i * UNROLL + u) * L, L)]
                        e = jnp.exp(d * inv_tau - inv_tau)
                        v_e = v_e + e
                        v_p = v_p + x * e
                    return (v_e, v_p)

                v_e, v_p = lax.fori_loop(
                    0, v // (L * UNROLL), col_body, (zero, zero))
                es_all[tok, :] = v_e
                dt_all[tok, :] = v_p

        issue(0, e0, x0, sem_e0, sem_x0)

        def pair_body(k, _):
            g0 = 2 * k
            drain(e0, x0, sem_e0, sem_x0)
            issue(g0 + 1, e1, x1, sem_e1, sem_x1)
            compute(g0, e0, x0)
            drain(e1, x1, sem_e1, sem_x1)
            issue(jnp.minimum(g0 + 2, ngp - 1), e0, x0, sem_e0, sem_x0)
            compute(g0 + 1, e1, x1)
            return 0

        lax.fori_loop(0, pairs, pair_body, 0)
        drain(e0, x0, sem_e0, sem_x0)

        pltpu.sync_copy(es_all, esum_hbm.at[pl.ds(base, tpw)])
        pltpu.sync_copy(dt_all, dot_hbm.at[pl.ds(base, tpw)])

    return sc_loss


GT_BLK = 8


@functools.lru_cache(maxsize=None)
def _build_tc_gt(bt: int, v: int):
    def gt_body(x_ref, t_ref, o_ref):
        cols = lax.broadcasted_iota(jnp.int32, (GT_BLK, v), 1)
        hit = cols == t_ref[...]
        o_ref[...] = jnp.sum(jnp.where(hit, x_ref[...], 0.0), axis=1,
                             keepdims=True)

    return pl.pallas_call(
        gt_body,
        grid=(bt // GT_BLK,),
        in_specs=[
            pl.BlockSpec((GT_BLK, v), lambda i: (i, 0)),
            pl.BlockSpec((GT_BLK, 1), lambda i: (i, 0)),
        ],
        out_specs=pl.BlockSpec((GT_BLK, 1), lambda i: (i, 0)),
        out_shape=jax.ShapeDtypeStruct((bt, 1), jnp.float32),
    )


def kernel(input, target, mask, pre_scores, Dist):
    b, t, v = input.shape
    bt = b * t
    x = input.reshape(bt, v)
    tgt = target.reshape(bt).astype(jnp.int32)
    tgt2 = tgt.reshape(bt // GP, GP)
    msk = mask.reshape(bt)
    esumv, dotv = _build_sc_loss(bt, v)(x, tgt2, Dist)
    gtv = _build_tc_gt(bt, v)(x, tgt.reshape(bt, 1))
    s_m = jnp.sum(msk)
    s_g = jnp.vdot(msk, gtv[:, 0])
    s_e = jnp.vdot(msk, jnp.sum(esumv, axis=1))
    s_p = jnp.vdot(msk, jnp.sum(dotv, axis=1))
    real = -s_g / s_m
    smooth = -s_p / s_e
    return (real, ALPHA * smooth + (1.0 - ALPHA) * real)


# speedup vs baseline: 1.0762x; 1.0762x over previous
import functools

import jax
import jax.numpy as jnp
from jax import lax
from jax.experimental import pallas as pl
from jax.experimental.pallas import tpu as pltpu
from jax.experimental.pallas import tpu_sc as plsc

TAU = 0.8
ALPHA = 0.3
NC, NS, L = 2, 16, 16
NW = NC * NS
GP = 4
UNROLL = 8


@functools.lru_cache(maxsize=None)
def _build_sc_loss(bt: int, v: int):
    tpw = bt // NW
    ngp = tpw // GP
    pairs = ngp // 2
    mesh = plsc.VectorSubcoreMesh(
        core_axis_name="c", subcore_axis_name="s",
        num_cores=NC, num_subcores=NS)

    @functools.partial(
        pl.kernel,
        out_type=(
            jax.ShapeDtypeStruct((bt, L), jnp.float32),
            jax.ShapeDtypeStruct((bt, L), jnp.float32),
        ),
        mesh=mesh,
        scratch_types=[
            pltpu.VMEM((ngp, GP), jnp.int32),
            pltpu.VMEM((GP, v), jnp.float32),
            pltpu.VMEM((GP, v), jnp.float32),
            pltpu.VMEM((GP * v,), jnp.float32),
            pltpu.VMEM((GP * v,), jnp.float32),
            pltpu.VMEM((tpw, L), jnp.float32),
            pltpu.VMEM((tpw, L), jnp.float32),
            pltpu.SemaphoreType.DMA,
            pltpu.SemaphoreType.DMA,
            pltpu.SemaphoreType.DMA,
            pltpu.SemaphoreType.DMA,
        ],
    )
    def sc_loss(x_hbm, tgt2_hbm, dist_hbm,
                esum_hbm, dot_hbm,
                tgt2_v, e0, e1, x0, x1, es_all, dt_all,
                sem_e0, sem_e1, sem_x0, sem_x1):
        wid = lax.axis_index("c") * NS + lax.axis_index("s")
        base = wid * tpw
        base_g = pl.multiple_of(wid * ngp, 8)
        inv_tau = 1.0 / TAU
        pltpu.sync_copy(tgt2_hbm.at[pl.ds(base_g, ngp)], tgt2_v)
        zero = jnp.zeros((L,), jnp.float32)

        def issue(g, ebuf, xbuf, sem_e, sem_x):
            pltpu.async_copy(dist_hbm.at[tgt2_v.at[g]], ebuf, sem_e)
            for r in range(GP):
                pltpu.async_copy(x_hbm.at[base + g * GP + r],
                                 xbuf.at[pl.ds(r * v, v)], sem_x)

        def drain(ebuf, xbuf, sem_e, sem_x):
            pltpu.make_async_copy(dist_hbm.at[tgt2_v.at[0]],
                                  ebuf, sem_e).wait()
            for r in range(GP):
                pltpu.make_async_copy(x_hbm.at[0],
                                      xbuf.at[pl.ds(r * v, v)], sem_x).wait()

        def compute(g, ebuf, xbuf):
            for r in range(GP):
                tok = g * GP + r

                def col_body(i, c3):
                    v_e, v_p = c3
                    for u in range(UNROLL):
                        d = ebuf[r, pl.ds((i * UNROLL + u) * L, L)]
                        x = xbuf[pl.ds(r * v + (i * UNROLL + u) * L, L)]
                        e = jnp.exp(d * inv_tau - inv_tau)
                        v_e = v_e + e
                        v_p = v_p + x * e
                    return (v_e, v_p)

                v_e, v_p = lax.fori_loop(
                    0, v // (L * UNROLL), col_body, (zero, zero))
                es_all[tok, :] = v_e
                dt_all[tok, :] = v_p

        issue(0, e0, x0, sem_e0, sem_x0)

        def pair_body(k, _):
            g0 = 2 * k
            drain(e0, x0, sem_e0, sem_x0)
            issue(g0 + 1, e1, x1, sem_e1, sem_x1)
            compute(g0, e0, x0)
            drain(e1, x1, sem_e1, sem_x1)
            issue(jnp.minimum(g0 + 2, ngp - 1), e0, x0, sem_e0, sem_x0)
            compute(g0 + 1, e1, x1)
            return 0

        lax.fori_loop(0, pairs, pair_body, 0)
        drain(e0, x0, sem_e0, sem_x0)

        pltpu.sync_copy(es_all, esum_hbm.at[pl.ds(base, tpw)])
        pltpu.sync_copy(dt_all, dot_hbm.at[pl.ds(base, tpw)])

    return sc_loss


GT_BLK = 8


@functools.lru_cache(maxsize=None)
def _build_tc_gt(bt: int, v: int):
    def gt_body(x_ref, t_ref, o_ref):
        col = lax.broadcasted_iota(jnp.int32, (1, 128), 1)
        for j in range(GT_BLK):
            c = t_ref[j, 0]
            chunk = x_ref[pl.ds(j, 1), pl.ds((c >> 7) * 128, 128)]
            o_ref[pl.ds(j, 1), :] = jnp.where(col == (c & 127), chunk, 0.0)

    return pl.pallas_call(
        gt_body,
        grid=(bt // GT_BLK,),
        in_specs=[
            pl.BlockSpec((GT_BLK, v), lambda i: (i, 0)),
            pl.BlockSpec((GT_BLK, 1), lambda i: (i, 0),
                         memory_space=pltpu.SMEM),
        ],
        out_specs=pl.BlockSpec((GT_BLK, 128), lambda i: (i, 0)),
        out_shape=jax.ShapeDtypeStruct((bt, 128), jnp.float32),
    )


def kernel(input, target, mask, pre_scores, Dist):
    b, t, v = input.shape
    bt = b * t
    x = input.reshape(bt, v)
    tgt = target.reshape(bt).astype(jnp.int32)
    tgt2 = tgt.reshape(bt // GP, GP)
    msk = mask.reshape(bt)
    esumv, dotv = _build_sc_loss(bt, v)(x, tgt2, Dist)
    gtv = _build_tc_gt(bt, v)(x, tgt.reshape(bt, 1))
    s_m = jnp.sum(msk)
    s_g = jnp.vdot(msk, jnp.sum(gtv, axis=1))
    s_e = jnp.vdot(msk, jnp.sum(esumv, axis=1))
    s_p = jnp.vdot(msk, jnp.sum(dotv, axis=1))
    real = -s_g / s_m
    smooth = -s_p / s_e
    return (real, ALPHA * smooth + (1.0 - ALPHA) * real)


# speedup vs baseline: 3.2643x; 3.0333x over previous
import functools

import jax
import jax.numpy as jnp
from jax import lax
from jax.experimental import pallas as pl
from jax.experimental.pallas import tpu as pltpu
from jax.experimental.pallas import tpu_sc as plsc

TAU = 0.8
ALPHA = 0.3
NC, NS, L = 2, 16, 16
NW = NC * NS
GP = 4
UNROLL = 8


@functools.lru_cache(maxsize=None)
def _build_sc_loss(bt: int, v: int):
    tpw = bt // NW
    ngp = tpw // GP
    pairs = ngp // 2
    mesh = plsc.VectorSubcoreMesh(
        core_axis_name="c", subcore_axis_name="s",
        num_cores=NC, num_subcores=NS)

    @functools.partial(
        pl.kernel,
        out_type=(
            jax.ShapeDtypeStruct((bt * L,), jnp.float32),
            jax.ShapeDtypeStruct((bt * L,), jnp.float32),
            jax.ShapeDtypeStruct((bt * L,), jnp.float32),
        ),
        mesh=mesh,
        scratch_types=[
            pltpu.VMEM((ngp, GP), jnp.int32),
            pltpu.VMEM((tpw * L,), jnp.int32),
            pltpu.VMEM((GP, v), jnp.float32),
            pltpu.VMEM((GP, v), jnp.float32),
            pltpu.VMEM((GP * v,), jnp.float32),
            pltpu.VMEM((GP * v,), jnp.float32),
            pltpu.VMEM((tpw * L,), jnp.float32),
            pltpu.VMEM((tpw * L,), jnp.float32),
            pltpu.VMEM((tpw * L,), jnp.float32),
            pltpu.SemaphoreType.DMA,
            pltpu.SemaphoreType.DMA,
            pltpu.SemaphoreType.DMA,
            pltpu.SemaphoreType.DMA,
        ],
    )
    def sc_loss(x_hbm, tgt2_hbm, tgtb_hbm, dist_hbm,
                esum_hbm, dot_hbm, gt_hbm,
                tgt2_v, tgtb_v, e0, e1, x0, x1, es_all, dt_all, gt_all,
                sem_e0, sem_e1, sem_x0, sem_x1):
        wid = lax.axis_index("c") * NS + lax.axis_index("s")
        base = wid * tpw
        base_g = pl.multiple_of(wid * ngp, 8)
        inv_tau = 1.0 / TAU
        pltpu.sync_copy(tgt2_hbm.at[pl.ds(base_g, ngp)], tgt2_v)
        pltpu.sync_copy(tgtb_hbm.at[pl.ds(base * L, tpw * L)], tgtb_v)
        zero = jnp.zeros((L,), jnp.float32)
        lane = lax.broadcasted_iota(jnp.int32, (L,), 0)

        def issue(g, ebuf, xbuf, sem_e, sem_x):
            pltpu.async_copy(dist_hbm.at[tgt2_v.at[g]], ebuf, sem_e)
            for r in range(GP):
                pltpu.async_copy(x_hbm.at[base + g * GP + r],
                                 xbuf.at[pl.ds(r * v, v)], sem_x)

        def drain(ebuf, xbuf, sem_e, sem_x):
            pltpu.make_async_copy(dist_hbm.at[tgt2_v.at[0]],
                                  ebuf, sem_e).wait()
            for r in range(GP):
                pltpu.make_async_copy(x_hbm.at[0],
                                      xbuf.at[pl.ds(r * v, v)], sem_x).wait()

        def compute(g, ebuf, xbuf):
            for r in range(GP):
                tok = g * GP + r
                tvec = tgtb_v[pl.ds(tok * L, L)]
                comb = jnp.where(lane == (tvec & (L - 1)), tvec >> 4, -1)

                def col_body(i, c3):
                    v_e, v_p, v_g = c3
                    for u in range(UNROLL):
                        iu = i * UNROLL + u
                        d = ebuf[r, pl.ds(iu * L, L)]
                        x = xbuf[pl.ds(r * v + iu * L, L)]
                        e = jnp.exp(d * inv_tau - inv_tau)
                        v_e = v_e + e
                        v_p = v_p + x * e
                        v_g = jnp.where(comb == iu, x, v_g)
                    return (v_e, v_p, v_g)

                v_e, v_p, v_g = lax.fori_loop(
                    0, v // (L * UNROLL), col_body, (zero, zero, zero))
                es_all[pl.ds(tok * L, L)] = v_e
                dt_all[pl.ds(tok * L, L)] = v_p
                gt_all[pl.ds(tok * L, L)] = v_g

        issue(0, e0, x0, sem_e0, sem_x0)

        def pair_body(k, _):
            g0 = 2 * k
            drain(e0, x0, sem_e0, sem_x0)
            issue(g0 + 1, e1, x1, sem_e1, sem_x1)
            compute(g0, e0, x0)
            drain(e1, x1, sem_e1, sem_x1)
            issue(jnp.minimum(g0 + 2, ngp - 1), e0, x0, sem_e0, sem_x0)
            compute(g0 + 1, e1, x1)
            return 0

        lax.fori_loop(0, pairs, pair_body, 0)
        drain(e0, x0, sem_e0, sem_x0)

        pltpu.sync_copy(es_all, esum_hbm.at[pl.ds(base * L, tpw * L)])
        pltpu.sync_copy(dt_all, dot_hbm.at[pl.ds(base * L, tpw * L)])
        pltpu.sync_copy(gt_all, gt_hbm.at[pl.ds(base * L, tpw * L)])

    return sc_loss


def kernel(input, target, mask, pre_scores, Dist):
    b, t, v = input.shape
    bt = b * t
    x = input.reshape(bt, v)
    tgt = target.reshape(bt).astype(jnp.int32)
    tgt2 = tgt.reshape(bt // GP, GP)
    tgtb = jnp.broadcast_to(tgt[:, None], (bt, L)).reshape(bt * L)
    msk = mask.reshape(bt)
    esumv, dotv, gtv = _build_sc_loss(bt, v)(x, tgt2, tgtb, Dist)
    s_m = jnp.sum(msk)
    s_g = jnp.vdot(msk, jnp.sum(gtv.reshape(bt, L), axis=1))
    s_e = jnp.vdot(msk, jnp.sum(esumv.reshape(bt, L), axis=1))
    s_p = jnp.vdot(msk, jnp.sum(dotv.reshape(bt, L), axis=1))
    real = -s_g / s_m
    smooth = -s_p / s_e
    return (real, ALPHA * smooth + (1.0 - ALPHA) * real)


# speedup vs baseline: 3.4052x; 1.0432x over previous
import functools

import jax
import jax.numpy as jnp
from jax import lax
from jax.experimental import pallas as pl
from jax.experimental.pallas import tpu as pltpu
from jax.experimental.pallas import tpu_sc as plsc

TAU = 0.8
ALPHA = 0.3
NC, NS, L = 2, 16, 16
NW = NC * NS
GP = 4
UNROLL = 8


@functools.lru_cache(maxsize=None)
def _build_sc_loss(bt: int, v: int):
    tpw = bt // NW
    ngp = tpw // GP
    pairs = ngp // 2
    mesh = plsc.VectorSubcoreMesh(
        core_axis_name="c", subcore_axis_name="s",
        num_cores=NC, num_subcores=NS)

    @functools.partial(
        pl.kernel,
        out_type=jax.ShapeDtypeStruct((NW, 3, L), jnp.float32),
        mesh=mesh,
        scratch_types=[
            pltpu.VMEM((ngp, GP), jnp.int32),
            pltpu.VMEM((tpw * L,), jnp.int32),
            pltpu.VMEM((tpw * L,), jnp.float32),
            pltpu.VMEM((GP, v), jnp.float32),
            pltpu.VMEM((GP, v), jnp.float32),
            pltpu.VMEM((GP * v,), jnp.float32),
            pltpu.VMEM((GP * v,), jnp.float32),
            pltpu.VMEM((3, L), jnp.float32),
            pltpu.SemaphoreType.DMA,
            pltpu.SemaphoreType.DMA,
            pltpu.SemaphoreType.DMA,
            pltpu.SemaphoreType.DMA,
        ],
    )
    def sc_loss(x_hbm, tgt2_hbm, tgtb_hbm, mskb_hbm, dist_hbm, out_hbm,
                tgt2_v, tgtb_v, mskb_v, e0, e1, x0, x1, st,
                sem_e0, sem_e1, sem_x0, sem_x1):
        wid = lax.axis_index("c") * NS + lax.axis_index("s")
        base = wid * tpw
        base_g = pl.multiple_of(wid * ngp, 8)
        inv_tau = 1.0 / TAU
        pltpu.sync_copy(tgt2_hbm.at[pl.ds(base_g, ngp)], tgt2_v)
        pltpu.sync_copy(tgtb_hbm.at[pl.ds(base * L, tpw * L)], tgtb_v)
        pltpu.sync_copy(mskb_hbm.at[pl.ds(base * L, tpw * L)], mskb_v)
        zero = jnp.zeros((L,), jnp.float32)
        lane = lax.broadcasted_iota(jnp.int32, (L,), 0)

        def issue(g, ebuf, xbuf, sem_e, sem_x):
            pltpu.async_copy(dist_hbm.at[tgt2_v.at[g]], ebuf, sem_e)
            for r in range(GP):
                pltpu.async_copy(x_hbm.at[base + g * GP + r],
                                 xbuf.at[pl.ds(r * v, v)], sem_x)

        def drain(ebuf, xbuf, sem_e, sem_x):
            pltpu.make_async_copy(dist_hbm.at[tgt2_v.at[0]],
                                  ebuf, sem_e).wait()
            for r in range(GP):
                pltpu.make_async_copy(x_hbm.at[0],
                                      xbuf.at[pl.ds(r * v, v)], sem_x).wait()

        def compute(g, ebuf, xbuf, accs):
            acc_e, acc_p, acc_g = accs
            for r in range(GP):
                tok = g * GP + r
                tvec = tgtb_v[pl.ds(tok * L, L)]
                mvec = mskb_v[pl.ds(tok * L, L)]
                comb = jnp.where(lane == (tvec & (L - 1)), tvec >> 4, -1)

                def col_body(i, c3):
                    v_e, v_p, v_g = c3
                    for u in range(UNROLL):
                        iu = i * UNROLL + u
                        d = ebuf[r, pl.ds(iu * L, L)]
                        x = xbuf[pl.ds(r * v + iu * L, L)]
                        e = jnp.exp(d * inv_tau - inv_tau)
                        v_e = v_e + e
                        v_p = v_p + x * e
                        v_g = jnp.where(comb == iu, x, v_g)
                    return (v_e, v_p, v_g)

                v_e, v_p, v_g = lax.fori_loop(
                    0, v // (L * UNROLL), col_body, (zero, zero, zero))
                acc_e = acc_e + v_e * mvec
                acc_p = acc_p + v_p * mvec
                acc_g = acc_g + v_g * mvec
            return (acc_e, acc_p, acc_g)

        issue(0, e0, x0, sem_e0, sem_x0)

        def pair_body(k, accs):
            g0 = 2 * k
            drain(e0, x0, sem_e0, sem_x0)
            issue(g0 + 1, e1, x1, sem_e1, sem_x1)
            accs = compute(g0, e0, x0, accs)
            drain(e1, x1, sem_e1, sem_x1)
            issue(jnp.minimum(g0 + 2, ngp - 1), e0, x0, sem_e0, sem_x0)
            accs = compute(g0 + 1, e1, x1, accs)
            return accs

        acc_e, acc_p, acc_g = lax.fori_loop(
            0, pairs, pair_body, (zero, zero, zero))
        drain(e0, x0, sem_e0, sem_x0)

        st[0, :] = acc_e
        st[1, :] = acc_p
        st[2, :] = acc_g
        pltpu.sync_copy(st, out_hbm.at[wid])

    return sc_loss


def kernel(input, target, mask, pre_scores, Dist):
    b, t, v = input.shape
    bt = b * t
    x = input.reshape(bt, v)
    tgt = target.reshape(bt).astype(jnp.int32)
    tgt2 = tgt.reshape(bt // GP, GP)
    tgtb = jnp.broadcast_to(tgt[:, None], (bt, L)).reshape(bt * L)
    mskb = jnp.broadcast_to(mask.reshape(bt)[:, None], (bt, L)).reshape(bt * L)
    msk = mask.reshape(bt)
    parts = _build_sc_loss(bt, v)(x, tgt2, tgtb, mskb, Dist)
    s = jnp.sum(parts, axis=(0, 2))
    s_m = jnp.sum(msk)
    real = -s[2] / s_m
    smooth = -s[1] / s[0]
    return (real, ALPHA * smooth + (1.0 - ALPHA) * real)
